# Initial kernel scaffold; baseline (speedup 1.0000x reference)
import jax
import jax.numpy as jnp
from jax.experimental import pallas as pl


def kernel(x, edge_index, t1_W, t1_b, t2_W, t2_b, gate_W, gate_b):
    raise NotImplementedError("write your pallas kernel here")


# speedup vs baseline: 4.5009x; 4.5009x over previous
import functools

import jax
import jax.numpy as jnp
from jax import lax
from jax.experimental import pallas as pl
from jax.experimental.pallas import tpu as pltpu
from jax.experimental.pallas import tpu_sc as plsc

N = 10000
D = 128
H = 64
C = 16
EPS = 0.3

NPAD = 10240
DUMMY = NPAD - 1
NW = 32
CHUNK = 128
ROWS_PER_TILE = NPAD // NW * 2

_f32 = jnp.float32


def _mesh():
    return plsc.VectorSubcoreMesh(core_axis_name="c", subcore_axis_name="s")


_SC_PARAMS = pltpu.CompilerParams(
    needs_layout_passes=False, use_tc_tiling_on_sc=False)


def _sc_deg(row3d, ch):
    @functools.partial(
        pl.kernel,
        out_type=jax.ShapeDtypeStruct((2, NPAD), _f32),
        mesh=_mesh(),
        compiler_params=_SC_PARAMS,
        scratch_types=[
            pltpu.VMEM((ch, CHUNK), jnp.int32),
            pltpu.VMEM((CHUNK,), _f32),
            pltpu.VMEM((ROWS_PER_TILE,), _f32),
            pltpu.VMEM_SHARED((NPAD,), _f32),
            pltpu.SemaphoreType.DMA,
        ],
    )
    def k(row_hbm, out_hbm, idx_v, ones_v, z_v, deg_sh, sem):
        c = lax.axis_index("c")
        s = lax.axis_index("s")
        wid = c * 16 + s
        for i in range(CHUNK // 16):
            ones_v[pl.ds(i * 16, 16)] = jnp.full((16,), 1.0, _f32)
        for i in range(ROWS_PER_TILE // 16):
            z_v[pl.ds(i * 16, 16)] = jnp.zeros((16,), _f32)
        pltpu.sync_copy(z_v, deg_sh.at[pl.ds(s * ROWS_PER_TILE, ROWS_PER_TILE)])
        pltpu.sync_copy(row_hbm.at[wid], idx_v)
        plsc.subcore_barrier()

        def body(j, carry):
            pltpu.sync_copy(ones_v, deg_sh.at[idx_v.at[j]], add=True)
            return carry

        lax.fori_loop(0, ch, body, 0)
        plsc.subcore_barrier()
        pltpu.sync_copy(
            deg_sh.at[pl.ds(s * ROWS_PER_TILE, ROWS_PER_TILE)],
            out_hbm.at[c, pl.ds(s * ROWS_PER_TILE, ROWS_PER_TILE)],
        )

    return k(row3d)


def _sc_edge(h, S, row3d, col3d, zeros_big, ch):
    @functools.partial(
        pl.kernel,
        out_type=jax.ShapeDtypeStruct((2, NPAD, H), _f32),
        mesh=_mesh(),
        compiler_params=_SC_PARAMS,
        scratch_types=[
            pltpu.VMEM((ch, CHUNK), jnp.int32),
            pltpu.VMEM((ch, CHUNK), jnp.int32),
            pltpu.VMEM((NPAD,), _f32),
            pltpu.VMEM((NPAD,), _f32),
            pltpu.VMEM((NPAD,), _f32),
            pltpu.VMEM((CHUNK, H), _f32),
            pltpu.VMEM_SHARED((NPAD, H), _f32),
            pltpu.SemaphoreType.DMA,
        ],
    )
    def k(h_hbm, s_hbm, row_hbm, col_hbm, z_hbm, out_hbm,
          rowi, coli, atab, btab, ndtab, hrows, agg_sh, sem):
        c = lax.axis_index("c")
        s = lax.axis_index("s")
        wid = c * 16 + s
        pltpu.sync_copy(s_hbm.at[0], atab)
        pltpu.sync_copy(s_hbm.at[1], btab)
        pltpu.sync_copy(s_hbm.at[2], ndtab)
        pltpu.sync_copy(row_hbm.at[wid], rowi)
        pltpu.sync_copy(col_hbm.at[wid], coli)
        pltpu.sync_copy(
            z_hbm.at[pl.ds(s * ROWS_PER_TILE, ROWS_PER_TILE)],
            agg_sh.at[pl.ds(s * ROWS_PER_TILE, ROWS_PER_TILE)],
        )
        plsc.subcore_barrier()

        def chunk(j, carry):
            pltpu.async_copy(h_hbm.at[rowi.at[j]], hrows, sem).wait()
            for i in range(CHUNK // 16):
                r16 = rowi[j, pl.ds(i * 16, 16)]
                c16 = coli[j, pl.ds(i * 16, 16)]
                ag = plsc.load_gather(atab, [r16])
                bg = plsc.load_gather(btab, [c16])
                nr = plsc.load_gather(ndtab, [r16])
                nc = plsc.load_gather(ndtab, [c16])
                t = ag + bg
                sg = jnp.sign(t)
                u = jnp.exp(-2.0 * jnp.abs(t))
                th = sg * (1.0 - u) / (1.0 + u)
                nv = th * nr * nc
                epos = lax.iota(jnp.int32, 16) + i * 16
                for q in range(H):
                    cq = jnp.full((16,), q, jnp.int32)
                    v = plsc.load_gather(hrows, [epos, cq])
                    plsc.store_scatter(hrows, [epos, cq], v * nv)
            pltpu.sync_copy(hrows, agg_sh.at[coli.at[j]], add=True)
            return carry

        lax.fori_loop(0, ch, chunk, 0)
        plsc.subcore_barrier()
        pltpu.sync_copy(
            agg_sh.at[pl.ds(s * ROWS_PER_TILE, ROWS_PER_TILE)],
            out_hbm.at[c, pl.ds(s * ROWS_PER_TILE, ROWS_PER_TILE)],
        )

    return k(h, S, row3d, col3d, zeros_big)


_DN = (((1,), (1,)), ((), ()))
_PREC = lax.Precision.HIGHEST
BT = 2048


def _tc_pre(x_pad, t1_W, t1_b2, G8, gb, deg2):
    def body(x_ref, w_ref, b_ref, g_ref, gb_ref, d_ref, h_ref, s_ref):
        xb = x_ref[...]
        hv = lax.dot_general(xb, w_ref[...], _DN, precision=_PREC) + b_ref[...]
        hv = jnp.maximum(hv, 0.0)
        h_ref[...] = hv
        sdot = lax.dot_general(g_ref[...], hv, _DN, precision=_PREC)
        deg = jnp.maximum(d_ref[0, :] + d_ref[1, :], 1.0)
        nd = lax.rsqrt(deg)
        ri = lax.broadcasted_iota(jnp.int32, (8, BT), 0)
        sdot = sdot + jnp.where(ri == 0, gb_ref[0, 0], 0.0)
        sdot = sdot + jnp.where(ri == 2, jnp.broadcast_to(nd[None, :], (8, BT)), 0.0)
        s_ref[...] = sdot

    return pl.pallas_call(
        body,
        grid=(NPAD // BT,),
        in_specs=[
            pl.BlockSpec((BT, D), lambda i: (i, 0)),
            pl.BlockSpec((H, D), lambda i: (0, 0)),
            pl.BlockSpec((1, H), lambda i: (0, 0)),
            pl.BlockSpec((8, H), lambda i: (0, 0)),
            pl.BlockSpec((1, 1), lambda i: (0, 0)),
            pl.BlockSpec((2, BT), lambda i: (0, i)),
        ],
        out_specs=[
            pl.BlockSpec((BT, H), lambda i: (i, 0)),
            pl.BlockSpec((8, BT), lambda i: (0, i)),
        ],
        out_shape=[
            jax.ShapeDtypeStruct((NPAD, H), _f32),
            jax.ShapeDtypeStruct((8, NPAD), _f32),
        ],
    )(x_pad, t1_W, t1_b2, G8, gb, deg2)


def _tc_mid(agg, h0, G8, gb, deg2):
    def body(a_ref, h0_ref, g_ref, gb_ref, d_ref, h_ref, s_ref):
        hv = EPS * h0_ref[...] + a_ref[0] + a_ref[1]
        h_ref[...] = hv
        sdot = lax.dot_general(g_ref[...], hv, _DN, precision=_PREC)
        deg = jnp.maximum(d_ref[0, :] + d_ref[1, :], 1.0)
        nd = lax.rsqrt(deg)
        ri = lax.broadcasted_iota(jnp.int32, (8, BT), 0)
        sdot = sdot + jnp.where(ri == 0, gb_ref[0, 0], 0.0)
        sdot = sdot + jnp.where(ri == 2, jnp.broadcast_to(nd[None, :], (8, BT)), 0.0)
        s_ref[...] = sdot

    return pl.pallas_call(
        body,
        grid=(NPAD // BT,),
        in_specs=[
            pl.BlockSpec((2, BT, H), lambda i: (0, i, 0)),
            pl.BlockSpec((BT, H), lambda i: (i, 0)),
            pl.BlockSpec((8, H), lambda i: (0, 0)),
            pl.BlockSpec((1, 1), lambda i: (0, 0)),
            pl.BlockSpec((2, BT), lambda i: (0, i)),
        ],
        out_specs=[
            pl.BlockSpec((BT, H), lambda i: (i, 0)),
            pl.BlockSpec((8, BT), lambda i: (0, i)),
        ],
        out_shape=[
            jax.ShapeDtypeStruct((NPAD, H), _f32),
            jax.ShapeDtypeStruct((8, NPAD), _f32),
        ],
    )(agg, h0, G8, gb, deg2)


BF = 2000


def _tc_final(agg, h0, t2_W, t2_b2):
    def body(a_ref, h0_ref, w_ref, b_ref, o_ref):
        hv = EPS * h0_ref[...] + a_ref[0] + a_ref[1]
        o = lax.dot_general(hv, w_ref[...], _DN, precision=_PREC) + b_ref[...]
        m = jnp.max(o, axis=1, keepdims=True)
        z = o - m
        lse = jnp.log(jnp.sum(jnp.exp(z), axis=1, keepdims=True))
        o_ref[...] = z - lse

    return pl.pallas_call(
        body,
        grid=(N // BF,),
        in_specs=[
            pl.BlockSpec((2, BF, H), lambda i: (0, i, 0)),
            pl.BlockSpec((BF, H), lambda i: (i, 0)),
            pl.BlockSpec((C, H), lambda i: (0, 0)),
            pl.BlockSpec((1, C), lambda i: (0, 0)),
        ],
        out_specs=pl.BlockSpec((BF, C), lambda i: (i, 0)),
        out_shape=jax.ShapeDtypeStruct((N, C), _f32),
    )(agg, h0, t2_W, t2_b2)


def kernel(x, edge_index, t1_W, t1_b, t2_W, t2_b, gate_W, gate_b):
    E = edge_index.shape[1]
    ept = -(-E // NW)
    ch = -(-ept // CHUNK)
    EP = NW * ch * CHUNK

    row = jnp.pad(edge_index[0], (0, EP - E), constant_values=DUMMY)
    col = jnp.pad(edge_index[1], (0, EP - E), constant_values=DUMMY)
    row3d = row.reshape(NW, ch, CHUNK)
    col3d = col.reshape(NW, ch, CHUNK)

    x_pad = jnp.pad(x, ((0, NPAD - x.shape[0]), (0, 0)))
    t1_b2 = t1_b.reshape(1, H)
    t2_b2 = t2_b.reshape(1, C)
    G80 = jnp.zeros((8, H), _f32).at[0].set(gate_W[0, :H]).at[1].set(gate_W[0, H:])
    G81 = jnp.zeros((8, H), _f32).at[0].set(gate_W[1, :H]).at[1].set(gate_W[1, H:])
    gb0 = gate_b[0].reshape(1, 1)
    gb1 = gate_b[1].reshape(1, 1)
    zeros_big = jnp.zeros((NPAD, H), _f32)

    deg2 = _sc_deg(row3d, ch)
    h0, S0 = _tc_pre(x_pad, t1_W, t1_b2, G80, gb0, deg2)
    agg = _sc_edge(h0, S0, row3d, col3d, zeros_big, ch)
    h1, S1 = _tc_mid(agg, h0, G81, gb1, deg2)
    agg2 = _sc_edge(h1, S1, row3d, col3d, zeros_big, ch)
    return _tc_final(agg2, h0, t2_W, t2_b2)


# speedup vs baseline: 5.4157x; 1.2033x over previous
import functools

import jax
import jax.numpy as jnp
from jax import lax
from jax.experimental import pallas as pl
from jax.experimental.pallas import tpu as pltpu
from jax.experimental.pallas import tpu_sc as plsc

N = 10000
D = 128
H = 64
C = 16
EPS = 0.3

NPAD = 10240
DUMMY = NPAD - 1
NW = 32
CHUNK = 128
ROWS_PER_TILE = NPAD // NW * 2

_f32 = jnp.float32


def _mesh():
    return plsc.VectorSubcoreMesh(core_axis_name="c", subcore_axis_name="s")


_SC_PARAMS = pltpu.CompilerParams(
    needs_layout_passes=False, use_tc_tiling_on_sc=False)


def _sc_deg(row3d, ch):
    @functools.partial(
        pl.kernel,
        out_type=jax.ShapeDtypeStruct((2, NPAD), _f32),
        mesh=_mesh(),
        compiler_params=_SC_PARAMS,
        scratch_types=[
            pltpu.VMEM((ch, CHUNK), jnp.int32),
            pltpu.VMEM((CHUNK,), _f32),
            pltpu.VMEM((ROWS_PER_TILE,), _f32),
            pltpu.VMEM_SHARED((NPAD,), _f32),
            pltpu.SemaphoreType.DMA,
        ],
    )
    def k(row_hbm, out_hbm, idx_v, ones_v, z_v, deg_sh, sem):
        c = lax.axis_index("c")
        s = lax.axis_index("s")
        wid = c * 16 + s
        for i in range(CHUNK // 16):
            ones_v[pl.ds(i * 16, 16)] = jnp.full((16,), 1.0, _f32)
        for i in range(ROWS_PER_TILE // 16):
            z_v[pl.ds(i * 16, 16)] = jnp.zeros((16,), _f32)
        pltpu.sync_copy(z_v, deg_sh.at[pl.ds(s * ROWS_PER_TILE, ROWS_PER_TILE)])
        pltpu.sync_copy(row_hbm.at[wid], idx_v)
        plsc.subcore_barrier()

        def body(j, carry):
            pltpu.sync_copy(ones_v, deg_sh.at[idx_v.at[j]], add=True)
            return carry

        lax.fori_loop(0, ch, body, 0)
        plsc.subcore_barrier()
        pltpu.sync_copy(
            deg_sh.at[pl.ds(s * ROWS_PER_TILE, ROWS_PER_TILE)],
            out_hbm.at[c, pl.ds(s * ROWS_PER_TILE, ROWS_PER_TILE)],
        )

    return k(row3d)


def _sc_edge(h, S, row3d, col3d, zeros_big, ch):
    @functools.partial(
        pl.kernel,
        out_type=jax.ShapeDtypeStruct((2, NPAD, H), _f32),
        mesh=_mesh(),
        compiler_params=_SC_PARAMS,
        scratch_types=[
            pltpu.VMEM((ch, CHUNK), jnp.int32),
            pltpu.VMEM((ch, CHUNK), jnp.int32),
            pltpu.VMEM((NPAD,), _f32),
            pltpu.VMEM((NPAD,), _f32),
            pltpu.VMEM((NPAD,), _f32),
            pltpu.VMEM((CHUNK, H), _f32),
            pltpu.VMEM((CHUNK, H), _f32),
            pltpu.VMEM((CHUNK, H), _f32),
            pltpu.VMEM((CHUNK, H), _f32),
            pltpu.VMEM_SHARED((NPAD, H), _f32),
            pltpu.SemaphoreType.DMA,
            pltpu.SemaphoreType.DMA,
            pltpu.SemaphoreType.DMA,
            pltpu.SemaphoreType.DMA,
        ],
    )
    def k(h_hbm, s_hbm, row_hbm, col_hbm, z_hbm, out_hbm,
          rowi, coli, atab, btab, ndtab, g0, g1, s0, s1, agg_sh,
          gsem0, gsem1, ssem0, ssem1):
        c = lax.axis_index("c")
        s = lax.axis_index("s")
        wid = c * 16 + s
        gbuf = (g0, g1)
        sbuf = (s0, s1)
        gsem = (gsem0, gsem1)
        ssem = (ssem0, ssem1)
        pltpu.sync_copy(s_hbm.at[0], atab)
        pltpu.sync_copy(s_hbm.at[1], btab)
        pltpu.sync_copy(s_hbm.at[2], ndtab)
        pltpu.sync_copy(row_hbm.at[wid], rowi)
        pltpu.sync_copy(col_hbm.at[wid], coli)
        pltpu.sync_copy(
            z_hbm.at[pl.ds(s * ROWS_PER_TILE, ROWS_PER_TILE)],
            agg_sh.at[pl.ds(s * ROWS_PER_TILE, ROWS_PER_TILE)],
        )
        plsc.subcore_barrier()

        pltpu.async_copy(h_hbm.at[rowi.at[0]], g0, gsem0)

        def pair(jj, carry):
            for b in range(2):
                j = 2 * jj + b
                @pl.when(j + 1 < ch)
                def _():
                    pltpu.async_copy(
                        h_hbm.at[rowi.at[j + 1]], gbuf[1 - b], gsem[1 - b])
                pltpu.make_async_copy(
                    h_hbm.at[rowi.at[j]], gbuf[b], gsem[b]).wait()
                @pl.when(jj >= 1)
                def _():
                    pltpu.make_async_copy(
                        sbuf[b], agg_sh.at[coli.at[j]], ssem[b]).wait()
                def group(i, carry2):
                    r16 = rowi[j, pl.ds(i * 16, 16)]
                    c16 = coli[j, pl.ds(i * 16, 16)]
                    ag = plsc.load_gather(atab, [r16])
                    bg = plsc.load_gather(btab, [c16])
                    nr = plsc.load_gather(ndtab, [r16])
                    nc = plsc.load_gather(ndtab, [c16])
                    t = ag + bg
                    sg = jnp.sign(t)
                    u = jnp.exp(-2.0 * jnp.abs(t))
                    th = sg * (1.0 - u) / (1.0 + u)
                    nv = th * nr * nc
                    epos = lax.iota(jnp.int32, 16) + i * 16
                    for q in range(H):
                        cq = jnp.full((16,), q, jnp.int32)
                        v = plsc.load_gather(gbuf[b], [epos, cq])
                        plsc.store_scatter(sbuf[b], [epos, cq], v * nv)
                    return carry2

                lax.fori_loop(0, CHUNK // 16, group, 0)
                pltpu.async_copy(
                    sbuf[b], agg_sh.at[coli.at[j]], ssem[b], add=True)
            return carry

        lax.fori_loop(0, ch // 2, pair, 0)
        for b in range(2):
            pltpu.make_async_copy(
                sbuf[b], agg_sh.at[coli.at[0]], ssem[b]).wait()
        plsc.subcore_barrier()
        pltpu.sync_copy(
            agg_sh.at[pl.ds(s * ROWS_PER_TILE, ROWS_PER_TILE)],
            out_hbm.at[c, pl.ds(s * ROWS_PER_TILE, ROWS_PER_TILE)],
        )

    return k(h, S, row3d, col3d, zeros_big)


_DN = (((1,), (1,)), ((), ()))
_PREC = lax.Precision.HIGHEST
BT = 2048


def _tc_pre(x_pad, t1_W, t1_b2, G8, gb, deg2):
    def body(x_ref, w_ref, b_ref, g_ref, gb_ref, d_ref, h_ref, s_ref):
        xb = x_ref[...]
        hv = lax.dot_general(xb, w_ref[...], _DN, precision=_PREC) + b_ref[...]
        hv = jnp.maximum(hv, 0.0)
        h_ref[...] = hv
        sdot = lax.dot_general(g_ref[...], hv, _DN, precision=_PREC)
        deg = jnp.maximum(d_ref[0, :] + d_ref[1, :], 1.0)
        nd = lax.rsqrt(deg)
        ri = lax.broadcasted_iota(jnp.int32, (8, BT), 0)
        sdot = sdot + jnp.where(ri == 0, gb_ref[0, 0], 0.0)
        sdot = sdot + jnp.where(ri == 2, jnp.broadcast_to(nd[None, :], (8, BT)), 0.0)
        s_ref[...] = sdot

    return pl.pallas_call(
        body,
        grid=(NPAD // BT,),
        in_specs=[
            pl.BlockSpec((BT, D), lambda i: (i, 0)),
            pl.BlockSpec((H, D), lambda i: (0, 0)),
            pl.BlockSpec((1, H), lambda i: (0, 0)),
            pl.BlockSpec((8, H), lambda i: (0, 0)),
            pl.BlockSpec((1, 1), lambda i: (0, 0)),
            pl.BlockSpec((2, BT), lambda i: (0, i)),
        ],
        out_specs=[
            pl.BlockSpec((BT, H), lambda i: (i, 0)),
            pl.BlockSpec((8, BT), lambda i: (0, i)),
        ],
        out_shape=[
            jax.ShapeDtypeStruct((NPAD, H), _f32),
            jax.ShapeDtypeStruct((8, NPAD), _f32),
        ],
    )(x_pad, t1_W, t1_b2, G8, gb, deg2)


def _tc_mid(agg, h0, G8, gb, deg2):
    def body(a_ref, h0_ref, g_ref, gb_ref, d_ref, h_ref, s_ref):
        hv = EPS * h0_ref[...] + a_ref[0] + a_ref[1]
        h_ref[...] = hv
        sdot = lax.dot_general(g_ref[...], hv, _DN, precision=_PREC)
        deg = jnp.maximum(d_ref[0, :] + d_ref[1, :], 1.0)
        nd = lax.rsqrt(deg)
        ri = lax.broadcasted_iota(jnp.int32, (8, BT), 0)
        sdot = sdot + jnp.where(ri == 0, gb_ref[0, 0], 0.0)
        sdot = sdot + jnp.where(ri == 2, jnp.broadcast_to(nd[None, :], (8, BT)), 0.0)
        s_ref[...] = sdot

    return pl.pallas_call(
        body,
        grid=(NPAD // BT,),
        in_specs=[
            pl.BlockSpec((2, BT, H), lambda i: (0, i, 0)),
            pl.BlockSpec((BT, H), lambda i: (i, 0)),
            pl.BlockSpec((8, H), lambda i: (0, 0)),
            pl.BlockSpec((1, 1), lambda i: (0, 0)),
            pl.BlockSpec((2, BT), lambda i: (0, i)),
        ],
        out_specs=[
            pl.BlockSpec((BT, H), lambda i: (i, 0)),
            pl.BlockSpec((8, BT), lambda i: (0, i)),
        ],
        out_shape=[
            jax.ShapeDtypeStruct((NPAD, H), _f32),
            jax.ShapeDtypeStruct((8, NPAD), _f32),
        ],
    )(agg, h0, G8, gb, deg2)


BF = 2000


def _tc_final(agg, h0, t2_W, t2_b2):
    def body(a_ref, h0_ref, w_ref, b_ref, o_ref):
        hv = EPS * h0_ref[...] + a_ref[0] + a_ref[1]
        o = lax.dot_general(hv, w_ref[...], _DN, precision=_PREC) + b_ref[...]
        m = jnp.max(o, axis=1, keepdims=True)
        z = o - m
        lse = jnp.log(jnp.sum(jnp.exp(z), axis=1, keepdims=True))
        o_ref[...] = z - lse

    return pl.pallas_call(
        body,
        grid=(N // BF,),
        in_specs=[
            pl.BlockSpec((2, BF, H), lambda i: (0, i, 0)),
            pl.BlockSpec((BF, H), lambda i: (i, 0)),
            pl.BlockSpec((C, H), lambda i: (0, 0)),
            pl.BlockSpec((1, C), lambda i: (0, 0)),
        ],
        out_specs=pl.BlockSpec((BF, C), lambda i: (i, 0)),
        out_shape=jax.ShapeDtypeStruct((N, C), _f32),
    )(agg, h0, t2_W, t2_b2)


def kernel(x, edge_index, t1_W, t1_b, t2_W, t2_b, gate_W, gate_b):
    E = edge_index.shape[1]
    ept = -(-E // NW)
    ch = -(-ept // CHUNK)
    ch += ch & 1
    EP = NW * ch * CHUNK

    row = jnp.pad(edge_index[0], (0, EP - E), constant_values=DUMMY)
    col = jnp.pad(edge_index[1], (0, EP - E), constant_values=DUMMY)
    row3d = row.reshape(NW, ch, CHUNK)
    col3d = col.reshape(NW, ch, CHUNK)

    x_pad = jnp.pad(x, ((0, NPAD - x.shape[0]), (0, 0)))
    t1_b2 = t1_b.reshape(1, H)
    t2_b2 = t2_b.reshape(1, C)
    G80 = jnp.zeros((8, H), _f32).at[0].set(gate_W[0, :H]).at[1].set(gate_W[0, H:])
    G81 = jnp.zeros((8, H), _f32).at[0].set(gate_W[1, :H]).at[1].set(gate_W[1, H:])
    gb0 = gate_b[0].reshape(1, 1)
    gb1 = gate_b[1].reshape(1, 1)
    zeros_big = jnp.zeros((NPAD, H), _f32)

    deg2 = _sc_deg(row3d, ch)
    h0, S0 = _tc_pre(x_pad, t1_W, t1_b2, G80, gb0, deg2)
    agg = _sc_edge(h0, S0, row3d, col3d, zeros_big, ch)
    h1, S1 = _tc_mid(agg, h0, G81, gb1, deg2)
    agg2 = _sc_edge(h1, S1, row3d, col3d, zeros_big, ch)
    return _tc_final(agg2, h0, t2_W, t2_b2)


# speedup vs baseline: 15.3535x; 2.8350x over previous
import functools

import jax
import jax.numpy as jnp
from jax import lax
from jax.experimental import pallas as pl
from jax.experimental.pallas import tpu as pltpu
from jax.experimental.pallas import tpu_sc as plsc

N = 10000
D = 128
H = 64
C = 16
EPS = 0.3

NPAD = 10240
DUMMY = NPAD - 1
NW = 32
CHUNK = 128
ROWS_PER_TILE = NPAD // NW * 2

_f32 = jnp.float32


def _mesh():
    return plsc.VectorSubcoreMesh(core_axis_name="c", subcore_axis_name="s")


_SC_PARAMS = pltpu.CompilerParams(
    needs_layout_passes=False, use_tc_tiling_on_sc=False)


def _sc_deg(row3d, ch):
    @functools.partial(
        pl.kernel,
        out_type=jax.ShapeDtypeStruct((2, NPAD), _f32),
        mesh=_mesh(),
        compiler_params=_SC_PARAMS,
        scratch_types=[
            pltpu.VMEM((ch, CHUNK), jnp.int32),
            pltpu.VMEM((CHUNK,), _f32),
            pltpu.VMEM((ROWS_PER_TILE,), _f32),
            pltpu.VMEM_SHARED((NPAD,), _f32),
            pltpu.SemaphoreType.DMA,
        ],
    )
    def k(row_hbm, out_hbm, idx_v, ones_v, z_v, deg_sh, sem):
        c = lax.axis_index("c")
        s = lax.axis_index("s")
        wid = c * 16 + s
        for i in range(CHUNK // 16):
            ones_v[pl.ds(i * 16, 16)] = jnp.full((16,), 1.0, _f32)
        for i in range(ROWS_PER_TILE // 16):
            z_v[pl.ds(i * 16, 16)] = jnp.zeros((16,), _f32)
        pltpu.sync_copy(z_v, deg_sh.at[pl.ds(s * ROWS_PER_TILE, ROWS_PER_TILE)])
        pltpu.sync_copy(row_hbm.at[wid], idx_v)
        plsc.subcore_barrier()

        def body(j, carry):
            pltpu.sync_copy(ones_v, deg_sh.at[idx_v.at[j]], add=True)
            return carry

        lax.fori_loop(0, ch, body, 0)
        plsc.subcore_barrier()
        pltpu.sync_copy(
            deg_sh.at[pl.ds(s * ROWS_PER_TILE, ROWS_PER_TILE)],
            out_hbm.at[c, pl.ds(s * ROWS_PER_TILE, ROWS_PER_TILE)],
        )

    return k(row3d)


def _sc_edge(h, S, row3d, col3d, zeros_big, ch):
    @functools.partial(
        pl.kernel,
        out_type=jax.ShapeDtypeStruct((2, NPAD, H), _f32),
        mesh=_mesh(),
        compiler_params=_SC_PARAMS,
        scratch_types=[
            pltpu.VMEM((ch, CHUNK), jnp.int32),
            pltpu.VMEM((ch, CHUNK), jnp.int32),
            pltpu.VMEM((NPAD,), _f32),
            pltpu.VMEM((NPAD,), _f32),
            pltpu.VMEM((NPAD,), _f32),
            pltpu.VMEM((CHUNK, H), _f32),
            pltpu.VMEM((CHUNK, H), _f32),
            pltpu.VMEM((CHUNK, H), _f32),
            pltpu.VMEM((CHUNK, H), _f32),
            pltpu.VMEM_SHARED((NPAD, H), _f32),
            pltpu.SemaphoreType.DMA,
            pltpu.SemaphoreType.DMA,
            pltpu.SemaphoreType.DMA,
            pltpu.SemaphoreType.DMA,
        ],
    )
    def k(h_hbm, s_hbm, row_hbm, col_hbm, z_hbm, out_hbm,
          rowi, coli, atab, btab, ndtab, g0, g1, s0, s1, agg_sh,
          gsem0, gsem1, ssem0, ssem1):
        c = lax.axis_index("c")
        s = lax.axis_index("s")
        wid = c * 16 + s
        gbuf = (g0, g1)
        sbuf = (s0, s1)
        gsem = (gsem0, gsem1)
        ssem = (ssem0, ssem1)
        pltpu.sync_copy(s_hbm.at[0], atab)
        pltpu.sync_copy(s_hbm.at[1], btab)
        pltpu.sync_copy(s_hbm.at[2], ndtab)
        pltpu.sync_copy(row_hbm.at[wid], rowi)
        pltpu.sync_copy(col_hbm.at[wid], coli)
        pltpu.sync_copy(
            z_hbm.at[pl.ds(s * ROWS_PER_TILE, ROWS_PER_TILE)],
            agg_sh.at[pl.ds(s * ROWS_PER_TILE, ROWS_PER_TILE)],
        )
        plsc.subcore_barrier()

        pltpu.async_copy(h_hbm.at[rowi.at[0]], g0, gsem0)

        def pair(jj, carry):
            for b in range(2):
                j = 2 * jj + b
                @pl.when(j + 1 < ch)
                def _():
                    pltpu.async_copy(
                        h_hbm.at[rowi.at[j + 1]], gbuf[1 - b], gsem[1 - b])
                pltpu.make_async_copy(
                    h_hbm.at[rowi.at[j]], gbuf[b], gsem[b]).wait()
                @pl.when(jj >= 1)
                def _():
                    pltpu.make_async_copy(
                        sbuf[b], agg_sh.at[coli.at[j]], ssem[b]).wait()
                for i in range(CHUNK // 16):
                    r16 = rowi[j, pl.ds(i * 16, 16)]
                    c16 = coli[j, pl.ds(i * 16, 16)]
                    ag = plsc.load_gather(atab, [r16])
                    bg = plsc.load_gather(btab, [c16])
                    nr = plsc.load_gather(ndtab, [r16])
                    nc = plsc.load_gather(ndtab, [c16])
                    t = ag + bg
                    sg = jnp.sign(t)
                    u = jnp.exp(-2.0 * jnp.abs(t))
                    th = sg * (1.0 - u) / (1.0 + u)
                    nv = th * nr * nc
                    for e in range(16):
                        be = jnp.take_along_axis(
                            nv, jnp.full((16,), e, jnp.int32), axis=0)
                        r = i * 16 + e
                        for q in range(H // 16):
                            sbuf[b][r, pl.ds(q * 16, 16)] = (
                                gbuf[b][r, pl.ds(q * 16, 16)] * be)
                pltpu.async_copy(
                    sbuf[b], agg_sh.at[coli.at[j]], ssem[b], add=True)
            return carry

        lax.fori_loop(0, ch // 2, pair, 0)
        for b in range(2):
            pltpu.make_async_copy(
                sbuf[b], agg_sh.at[coli.at[0]], ssem[b]).wait()
        plsc.subcore_barrier()
        pltpu.sync_copy(
            agg_sh.at[pl.ds(s * ROWS_PER_TILE, ROWS_PER_TILE)],
            out_hbm.at[c, pl.ds(s * ROWS_PER_TILE, ROWS_PER_TILE)],
        )

    return k(h, S, row3d, col3d, zeros_big)


_DN = (((1,), (1,)), ((), ()))
_PREC = lax.Precision.HIGHEST
BT = 2048


def _tc_pre(x_pad, t1_W, t1_b2, G8, gb, deg2):
    def body(x_ref, w_ref, b_ref, g_ref, gb_ref, d_ref, h_ref, s_ref):
        xb = x_ref[...]
        hv = lax.dot_general(xb, w_ref[...], _DN, precision=_PREC) + b_ref[...]
        hv = jnp.maximum(hv, 0.0)
        h_ref[...] = hv
        sdot = lax.dot_general(g_ref[...], hv, _DN, precision=_PREC)
        deg = jnp.maximum(d_ref[0, :] + d_ref[1, :], 1.0)
        nd = lax.rsqrt(deg)
        ri = lax.broadcasted_iota(jnp.int32, (8, BT), 0)
        sdot = sdot + jnp.where(ri == 0, gb_ref[0, 0], 0.0)
        sdot = sdot + jnp.where(ri == 2, jnp.broadcast_to(nd[None, :], (8, BT)), 0.0)
        s_ref[...] = sdot

    return pl.pallas_call(
        body,
        grid=(NPAD // BT,),
        in_specs=[
            pl.BlockSpec((BT, D), lambda i: (i, 0)),
            pl.BlockSpec((H, D), lambda i: (0, 0)),
            pl.BlockSpec((1, H), lambda i: (0, 0)),
            pl.BlockSpec((8, H), lambda i: (0, 0)),
            pl.BlockSpec((1, 1), lambda i: (0, 0)),
            pl.BlockSpec((2, BT), lambda i: (0, i)),
        ],
        out_specs=[
            pl.BlockSpec((BT, H), lambda i: (i, 0)),
            pl.BlockSpec((8, BT), lambda i: (0, i)),
        ],
        out_shape=[
            jax.ShapeDtypeStruct((NPAD, H), _f32),
            jax.ShapeDtypeStruct((8, NPAD), _f32),
        ],
    )(x_pad, t1_W, t1_b2, G8, gb, deg2)


def _tc_mid(agg, h0, G8, gb, deg2):
    def body(a_ref, h0_ref, g_ref, gb_ref, d_ref, h_ref, s_ref):
        hv = EPS * h0_ref[...] + a_ref[0] + a_ref[1]
        h_ref[...] = hv
        sdot = lax.dot_general(g_ref[...], hv, _DN, precision=_PREC)
        deg = jnp.maximum(d_ref[0, :] + d_ref[1, :], 1.0)
        nd = lax.rsqrt(deg)
        ri = lax.broadcasted_iota(jnp.int32, (8, BT), 0)
        sdot = sdot + jnp.where(ri == 0, gb_ref[0, 0], 0.0)
        sdot = sdot + jnp.where(ri == 2, jnp.broadcast_to(nd[None, :], (8, BT)), 0.0)
        s_ref[...] = sdot

    return pl.pallas_call(
        body,
        grid=(NPAD // BT,),
        in_specs=[
            pl.BlockSpec((2, BT, H), lambda i: (0, i, 0)),
            pl.BlockSpec((BT, H), lambda i: (i, 0)),
            pl.BlockSpec((8, H), lambda i: (0, 0)),
            pl.BlockSpec((1, 1), lambda i: (0, 0)),
            pl.BlockSpec((2, BT), lambda i: (0, i)),
        ],
        out_specs=[
            pl.BlockSpec((BT, H), lambda i: (i, 0)),
            pl.BlockSpec((8, BT), lambda i: (0, i)),
        ],
        out_shape=[
            jax.ShapeDtypeStruct((NPAD, H), _f32),
            jax.ShapeDtypeStruct((8, NPAD), _f32),
        ],
    )(agg, h0, G8, gb, deg2)


BF = 2000


def _tc_final(agg, h0, t2_W, t2_b2):
    def body(a_ref, h0_ref, w_ref, b_ref, o_ref):
        hv = EPS * h0_ref[...] + a_ref[0] + a_ref[1]
        o = lax.dot_general(hv, w_ref[...], _DN, precision=_PREC) + b_ref[...]
        m = jnp.max(o, axis=1, keepdims=True)
        z = o - m
        lse = jnp.log(jnp.sum(jnp.exp(z), axis=1, keepdims=True))
        o_ref[...] = z - lse

    return pl.pallas_call(
        body,
        grid=(N // BF,),
        in_specs=[
            pl.BlockSpec((2, BF, H), lambda i: (0, i, 0)),
            pl.BlockSpec((BF, H), lambda i: (i, 0)),
            pl.BlockSpec((C, H), lambda i: (0, 0)),
            pl.BlockSpec((1, C), lambda i: (0, 0)),
        ],
        out_specs=pl.BlockSpec((BF, C), lambda i: (i, 0)),
        out_shape=jax.ShapeDtypeStruct((N, C), _f32),
    )(agg, h0, t2_W, t2_b2)


def kernel(x, edge_index, t1_W, t1_b, t2_W, t2_b, gate_W, gate_b):
    E = edge_index.shape[1]
    ept = -(-E // NW)
    ch = -(-ept // CHUNK)
    ch += ch & 1
    EP = NW * ch * CHUNK

    row = jnp.pad(edge_index[0], (0, EP - E), constant_values=DUMMY)
    col = jnp.pad(edge_index[1], (0, EP - E), constant_values=DUMMY)
    row3d = row.reshape(NW, ch, CHUNK)
    col3d = col.reshape(NW, ch, CHUNK)

    x_pad = jnp.pad(x, ((0, NPAD - x.shape[0]), (0, 0)))
    t1_b2 = t1_b.reshape(1, H)
    t2_b2 = t2_b.reshape(1, C)
    G80 = jnp.zeros((8, H), _f32).at[0].set(gate_W[0, :H]).at[1].set(gate_W[0, H:])
    G81 = jnp.zeros((8, H), _f32).at[0].set(gate_W[1, :H]).at[1].set(gate_W[1, H:])
    gb0 = gate_b[0].reshape(1, 1)
    gb1 = gate_b[1].reshape(1, 1)
    zeros_big = jnp.zeros((NPAD, H), _f32)

    deg2 = _sc_deg(row3d, ch)
    h0, S0 = _tc_pre(x_pad, t1_W, t1_b2, G80, gb0, deg2)
    agg = _sc_edge(h0, S0, row3d, col3d, zeros_big, ch)
    h1, S1 = _tc_mid(agg, h0, G81, gb1, deg2)
    agg2 = _sc_edge(h1, S1, row3d, col3d, zeros_big, ch)
    return _tc_final(agg2, h0, t2_W, t2_b2)
